# Initial kernel scaffold; baseline (speedup 1.0000x reference)
import jax
import jax.numpy as jnp
from jax.experimental import pallas as pl


def kernel(x, edge_index, batch, W1, a_src1, a_dst1, b1, W2, a_src2, a_dst2, b2, W3, a_src3, a_dst3, b3):
    raise NotImplementedError("write your pallas kernel here")


# speedup vs baseline: 9.7761x; 9.7761x over previous
import functools

import jax
import jax.numpy as jnp
from jax import lax
from jax.experimental import pallas as pl
from jax.experimental.pallas import tpu as pltpu
from jax.experimental.pallas import tpu_sc as plsc

N = 50000
IN = 128
F = 64
G = 64
E = 800000
EREAL = E + N
NP = 50176
EP = 851968
NSUB = 16
NPS = NP // NSUB
EPS = EP // NSUB
EC = 128
NEC = EPS // EC
NCHK = 112
BN = 448
GRID = NP // BN

_f32 = jnp.float32
_i32 = jnp.int32


def _tc1_body(x_ref, w_ref, a_ref, h_ref, sd_ref):
    h = jnp.dot(x_ref[...], w_ref[...], preferred_element_type=_f32)
    h_ref[...] = jnp.concatenate([h, jnp.zeros((BN, IN - F), _f32)], axis=1)
    sd_ref[...] = jnp.dot(h, a_ref[...], preferred_element_type=_f32)


_tc1 = pl.pallas_call(
    _tc1_body,
    grid=(GRID,),
    in_specs=[
        pl.BlockSpec((BN, IN), lambda i: (i, 0)),
        pl.BlockSpec((IN, F), lambda i: (0, 0)),
        pl.BlockSpec((F, 2), lambda i: (0, 0)),
    ],
    out_specs=[
        pl.BlockSpec((BN, IN), lambda i: (i, 0)),
        pl.BlockSpec((BN, 2), lambda i: (i, 0)),
    ],
    out_shape=[
        jax.ShapeDtypeStruct((NP, IN), _f32),
        jax.ShapeDtypeStruct((NP, 2), _f32),
    ],
)


def _norm_body(ua_ref, ub_ref, dna_ref, dnb_ref, b_ref, w_ref, a_ref,
               h_ref, sd_ref):
    u = jnp.concatenate([ua_ref[...], ub_ref[...]], axis=1)
    dn = jnp.maximum(dna_ref[...] + dnb_ref[...], 1e-30)
    hin = jnp.maximum(u / dn + b_ref[...], 0.0)
    h = jnp.dot(hin, w_ref[...], preferred_element_type=_f32)
    h_ref[...] = jnp.concatenate([h, jnp.zeros((BN, IN - F), _f32)], axis=1)
    sd_ref[...] = jnp.dot(h, a_ref[...], preferred_element_type=_f32)


_tc_norm = pl.pallas_call(
    _norm_body,
    grid=(GRID,),
    in_specs=[
        pl.BlockSpec((BN, 32), lambda i: (i, 0)),
        pl.BlockSpec((BN, 32), lambda i: (i, 0)),
        pl.BlockSpec((BN, 1), lambda i: (i, 0)),
        pl.BlockSpec((BN, 1), lambda i: (i, 0)),
        pl.BlockSpec((1, F), lambda i: (0, 0)),
        pl.BlockSpec((F, F), lambda i: (0, 0)),
        pl.BlockSpec((F, 2), lambda i: (0, 0)),
    ],
    out_specs=[
        pl.BlockSpec((BN, IN), lambda i: (i, 0)),
        pl.BlockSpec((BN, 2), lambda i: (i, 0)),
    ],
    out_shape=[
        jax.ShapeDtypeStruct((NP, IN), _f32),
        jax.ShapeDtypeStruct((NP, 2), _f32),
    ],
)


_mesh = plsc.VectorSubcoreMesh(core_axis_name="c", subcore_axis_name="s")

EPW = EP // 32
NWC = EPW // EC


def _iota16():
    return lax.broadcasted_iota(_i32, (16,), 0)


def _weights_pass(wid, sd_v, srcp, dstp, src_v, dst_v, w_v, w_out, den_s):
    iota = _iota16()

    def chunk(k, carry):
        base = wid * EPW + k * EC
        pltpu.sync_copy(srcp.at[pl.ds(base, EC)], src_v)
        pltpu.sync_copy(dstp.at[pl.ds(base, EC)], dst_v)
        for j in range(EC // 16):
            s16 = src_v[pl.ds(j * 16, 16)]
            d16 = dst_v[pl.ds(j * 16, 16)]
            a_s = plsc.load_gather(sd_v, [s16 + s16])
            a_d = plsc.load_gather(sd_v, [d16 + d16 + 1])
            e = a_s + a_d
            e = jnp.where(e > 0.0, e, 0.2 * e)
            w = jnp.exp(e)
            gid = base + (iota + j * 16)
            w = jnp.where(gid < EREAL, w, 0.0)
            w_v[pl.ds(j * 16, 16)] = w
        pltpu.sync_copy(w_v, w_out.at[pl.ds(base, EC)])
        pltpu.sync_copy(w_v, den_s.at[dst_v], add=True)
        return carry

    lax.fori_loop(0, NWC, chunk, 0)


def _sc_w_body(sd, srcp, dstp, zvec, w_out, dna, dnb,
               sd_v, src_v, dst_v, w_v, dbuf, den_s):
    cid = lax.axis_index("c")
    sid = lax.axis_index("s")
    wid = cid * NSUB + sid

    pltpu.sync_copy(zvec, dbuf)

    def zchunk(q, carry):
        pltpu.sync_copy(dbuf, den_s.at[pl.ds(sid * NPS + q * NCHK, NCHK)])
        return carry

    lax.fori_loop(0, NPS // NCHK, zchunk, 0)
    pltpu.sync_copy(sd, sd_v)
    plsc.subcore_barrier()
    _weights_pass(wid, sd_v, srcp, dstp, src_v, dst_v, w_v, w_out, den_s)
    plsc.subcore_barrier()

    def ochunk(q, carry):
        nbase = sid * NPS + q * NCHK
        pltpu.sync_copy(den_s.at[pl.ds(nbase, NCHK)], dbuf)

        @pl.when(cid == 0)
        def _():
            pltpu.sync_copy(dbuf, dna.at[pl.ds(nbase, NCHK)])

        @pl.when(cid == 1)
        def _():
            pltpu.sync_copy(dbuf, dnb.at[pl.ds(nbase, NCHK)])
        return carry

    lax.fori_loop(0, NPS // NCHK, ochunk, 0)


_sc_w = functools.partial(
    pl.kernel,
    out_type=(
        jax.ShapeDtypeStruct((EP,), _f32),
        jax.ShapeDtypeStruct((NP,), _f32),
        jax.ShapeDtypeStruct((NP,), _f32),
    ),
    mesh=_mesh,
    scratch_types=[
        pltpu.VMEM((2 * NP,), _f32),
        pltpu.VMEM((EC,), _i32),
        pltpu.VMEM((EC,), _i32),
        pltpu.VMEM((EC,), _f32),
        pltpu.VMEM((NCHK,), _f32),
        pltpu.VMEM_SHARED((NP,), _f32),
    ],
    compiler_params=pltpu.CompilerParams(needs_layout_passes=False, use_tc_tiling_on_sc=False),
)(_sc_w_body)


def _rows_pass(sid, table_hbm, off, w_hbm, srcp, dstp, src_v, dst_v, w_v,
               rows_v, sc_v, u_s, sem):
    def chunk(k, carry):
        base = sid * EPS + k * EC
        pltpu.sync_copy(srcp.at[pl.ds(base, EC)], src_v)
        pltpu.sync_copy(dstp.at[pl.ds(base, EC)], dst_v)
        pltpu.sync_copy(w_hbm.at[pl.ds(base, EC)], w_v)
        pltpu.async_copy(table_hbm.at[src_v], rows_v, sem).wait()
        for j in range(EC // 16):
            w16 = w_v[pl.ds(j * 16, 16)]
            for t in range(16):
                m = j * 16 + t
                wspl = w16.at[jnp.full((16,), t, _i32)].get(
                    mode="promise_in_bounds")
                sc_v[m, pl.ds(0, 16)] = rows_v[m, pl.ds(off, 16)] * wspl
                sc_v[m, pl.ds(16, 16)] = (
                    rows_v[m, pl.ds(off + 16, 16)] * wspl)
        pltpu.sync_copy(sc_v, u_s.at[dst_v], add=True)
        return carry

    lax.fori_loop(0, NEC, chunk, 0)


def _sc_u_half(off, sid, hfull, w_hbm, srcp, dstp, u_out,
               src_v, dst_v, w_v, rows_v, sc_v, obuf, obuf1, u_s, sem):
    z16 = jnp.zeros((16,), _f32)
    for m in range(NCHK):
        obuf[m, pl.ds(0, 16)] = z16
        obuf[m, pl.ds(16, 16)] = z16

    def zchunk(q, carry):
        pltpu.sync_copy(obuf, u_s.at[pl.ds(sid * NPS + q * NCHK, NCHK), :])
        return carry

    lax.fori_loop(0, NPS // NCHK, zchunk, 0)
    plsc.subcore_barrier()
    _rows_pass(sid, hfull, off, w_hbm, srcp, dstp, src_v, dst_v, w_v,
               rows_v, sc_v, u_s, sem)
    plsc.subcore_barrier()

    def ochunk(q, carry):
        nbase = sid * NPS + q * NCHK
        pltpu.sync_copy(u_s.at[pl.ds(nbase, NCHK), :], obuf)
        for m in range(NCHK):
            obuf1[pl.ds(m * 32, 16)] = obuf[m, pl.ds(0, 16)]
            obuf1[pl.ds(m * 32 + 16, 16)] = obuf[m, pl.ds(16, 16)]
        pltpu.sync_copy(obuf1, u_out.at[pl.ds(nbase * 32, NCHK * 32)])
        return carry

    lax.fori_loop(0, NPS // NCHK, ochunk, 0)


def _make_sc_u(off):
    def body(hfull, w_hbm, srcp, dstp, u_out,
             src_v, dst_v, w_v, rows_v, sc_v, obuf, obuf1, u_s, sem):
        sid = lax.axis_index("s")
        _sc_u_half(off, sid, hfull, w_hbm, srcp, dstp, u_out,
                   src_v, dst_v, w_v, rows_v, sc_v, obuf, obuf1, u_s, sem)

    return functools.partial(
        pl.kernel,
        out_type=jax.ShapeDtypeStruct((NP * 32,), _f32),
        mesh=plsc.VectorSubcoreMesh(core_axis_name="c",
                                    subcore_axis_name="s", num_cores=1),
        scratch_types=[
            pltpu.VMEM((EC,), _i32),
            pltpu.VMEM((EC,), _i32),
            pltpu.VMEM((EC,), _f32),
            pltpu.VMEM((EC, IN), _f32),
            pltpu.VMEM((EC, 32), _f32),
            pltpu.VMEM((NCHK, 32), _f32),
            pltpu.VMEM((NCHK * 32,), _f32),
            pltpu.VMEM_SHARED((NP, 32), _f32),
            pltpu.SemaphoreType.DMA,
        ],
        compiler_params=pltpu.CompilerParams(needs_layout_passes=False, use_tc_tiling_on_sc=False),
        name=f"sc_u_{off}",
    )(body)


_sc_u_a = _make_sc_u(0)
_sc_u_b = _make_sc_u(32)


def _pool_pass(sid, table_hbm, batch_p, ones_p, dna, dnb,
               b_v, o_v, dn_v, dn2_v, rows_v, cnt_v, rcnt_v,
               pool_s, cnt_s):
    def cnt_chunk(k, carry):
        base = sid * NPS + k * NCHK
        pltpu.sync_copy(batch_p.at[pl.ds(base, NCHK)], b_v)
        pltpu.sync_copy(ones_p.at[pl.ds(base, NCHK)], o_v)
        pltpu.sync_copy(o_v, cnt_s.at[b_v], add=True)
        return carry

    lax.fori_loop(0, NPS // NCHK, cnt_chunk, 0)
    plsc.subcore_barrier()

    pltpu.sync_copy(cnt_s, cnt_v)
    for k in range(G // 16):
        cv = cnt_v[pl.ds(k * 16, 16)]
        rcnt_v[pl.ds(k * 16, 16)] = 1.0 / jnp.maximum(cv, 1.0)

    def row_chunk(k, carry):
        base = sid * NPS + k * NCHK
        pltpu.sync_copy(batch_p.at[pl.ds(base, NCHK)], b_v)
        pltpu.sync_copy(ones_p.at[pl.ds(base, NCHK)], o_v)
        pltpu.sync_copy(dna.at[pl.ds(base, NCHK)], dn_v)
        pltpu.sync_copy(dnb.at[pl.ds(base, NCHK)], dn2_v)
        pltpu.sync_copy(table_hbm.at[pl.ds(base, NCHK), :], rows_v)
        for j in range(NCHK // 16):
            b16 = b_v[pl.ds(j * 16, 16)]
            o16 = o_v[pl.ds(j * 16, 16)]
            d16 = dn_v[pl.ds(j * 16, 16)] + dn2_v[pl.ds(j * 16, 16)]
            rc = plsc.load_gather(rcnt_v, [b16])
            w = o16 * rc / jnp.maximum(d16, 1e-30)
            for t in range(16):
                m = j * 16 + t
                wspl = w.at[jnp.full((16,), t, _i32)].get(
                    mode="promise_in_bounds")
                rows_v[m, pl.ds(0, 16)] = rows_v[m, pl.ds(0, 16)] * wspl
                rows_v[m, pl.ds(16, 16)] = rows_v[m, pl.ds(16, 16)] * wspl
        pltpu.sync_copy(rows_v, pool_s.at[b_v], add=True)
        return carry

    lax.fori_loop(0, NPS // NCHK, row_chunk, 0)
    plsc.subcore_barrier()


def _sc_pool_half(sid, table, batch_p, ones_p, dna, dnb, zrows, zvec, p_out,
                  cnt_out, write_cnt, b_v, o_v, dn_v, dn2_v, rows_v, cnt_v,
                  rcnt_v, pv, pool_s, cnt_s):
    @pl.when(sid == 0)
    def _():
        pltpu.sync_copy(zrows, rows_v)
        pltpu.sync_copy(zvec, dn_v)
        pltpu.sync_copy(rows_v.at[pl.ds(0, G), :], pool_s)
        pltpu.sync_copy(dn_v.at[pl.ds(0, G)], cnt_s)
    plsc.subcore_barrier()
    _pool_pass(sid, table, batch_p, ones_p, dna, dnb, b_v, o_v, dn_v, dn2_v,
               rows_v, cnt_v, rcnt_v, pool_s, cnt_s)

    @pl.when(sid == 0)
    def _():
        pltpu.sync_copy(pool_s, pv)
        pltpu.sync_copy(pv, p_out)
        if write_cnt:
            pltpu.sync_copy(cnt_s, cnt_v)
            pltpu.sync_copy(cnt_v, cnt_out)


def _sc_pool_body(u3a, u3b, dna, dnb, batch_p, ones_p, zrows, zvec,
                  pa, pb, cnt_out,
                  b_v, o_v, dn_v, dn2_v, rows_v, cnt_v, rcnt_v, pv,
                  pool_s, cnt_s):
    cid = lax.axis_index("c")
    sid = lax.axis_index("s")

    @pl.when(cid == 0)
    def _():
        _sc_pool_half(sid, u3a, batch_p, ones_p, dna, dnb, zrows, zvec, pa,
                      cnt_out, True, b_v, o_v, dn_v, dn2_v, rows_v, cnt_v,
                      rcnt_v, pv, pool_s, cnt_s)

    @pl.when(cid == 1)
    def _():
        _sc_pool_half(sid, u3b, batch_p, ones_p, dna, dnb, zrows, zvec, pb,
                      cnt_out, False, b_v, o_v, dn_v, dn2_v, rows_v, cnt_v,
                      rcnt_v, pv, pool_s, cnt_s)


_sc_pool = functools.partial(
    pl.kernel,
    out_type=(
        jax.ShapeDtypeStruct((G, 32), _f32),
        jax.ShapeDtypeStruct((G, 32), _f32),
        jax.ShapeDtypeStruct((G,), _f32),
    ),
    mesh=_mesh,
    scratch_types=[
        pltpu.VMEM((NCHK,), _i32),
        pltpu.VMEM((NCHK,), _f32),
        pltpu.VMEM((NCHK,), _f32),
        pltpu.VMEM((NCHK,), _f32),
        pltpu.VMEM((NCHK, 32), _f32),
        pltpu.VMEM((G,), _f32),
        pltpu.VMEM((G,), _f32),
        pltpu.VMEM((G, 32), _f32),
        pltpu.VMEM_SHARED((G, 32), _f32),
        pltpu.VMEM_SHARED((G,), _f32),
    ],
    compiler_params=pltpu.CompilerParams(needs_layout_passes=False, use_tc_tiling_on_sc=False),
)(_sc_pool_body)


def kernel(x, edge_index, batch, W1, a_src1, a_dst1, b1,
           W2, a_src2, a_dst2, b2, W3, a_src3, a_dst3, b3):
    loop = jnp.arange(N, dtype=edge_index.dtype)
    srcp = jnp.zeros((EP,), _i32).at[:EREAL].set(
        jnp.concatenate([edge_index[0], loop]).astype(_i32))
    dstp = jnp.zeros((EP,), _i32).at[:EREAL].set(
        jnp.concatenate([edge_index[1], loop]).astype(_i32))
    x_p = jnp.zeros((NP, IN), _f32).at[:N].set(x)
    batch_p = jnp.zeros((NP,), _i32).at[:N].set(batch.astype(_i32))
    ones_p = (jnp.arange(NP) < N).astype(_f32)
    zrows = jnp.zeros((NCHK, 32), _f32)
    zvec = jnp.zeros((NCHK,), _f32)

    a1 = jnp.stack([a_src1, a_dst1], axis=1)
    a2 = jnp.stack([a_src2, a_dst2], axis=1)
    a3 = jnp.stack([a_src3, a_dst3], axis=1)

    hf, sd = _tc1(x_p, W1, a1)
    wv, da, db = _sc_w(sd.reshape(2 * NP), srcp, dstp, zvec)
    ua = _sc_u_a(hf, wv, srcp, dstp)
    ub = _sc_u_b(hf, wv, srcp, dstp)
    hf, sd = _tc_norm(ua.reshape(NP, 32), ub.reshape(NP, 32),
                      da.reshape(NP, 1), db.reshape(NP, 1),
                      b1.reshape(1, F), W2, a2)
    wv, da, db = _sc_w(sd.reshape(2 * NP), srcp, dstp, zvec)
    ua = _sc_u_a(hf, wv, srcp, dstp)
    ub = _sc_u_b(hf, wv, srcp, dstp)
    hf, sd = _tc_norm(ua.reshape(NP, 32), ub.reshape(NP, 32),
                      da.reshape(NP, 1), db.reshape(NP, 1),
                      b2.reshape(1, F), W3, a3)
    wv, da, db = _sc_w(sd.reshape(2 * NP), srcp, dstp, zvec)
    ua = _sc_u_a(hf, wv, srcp, dstp)
    ub = _sc_u_b(hf, wv, srcp, dstp)
    pa, pb, cnt = _sc_pool(ua.reshape(NP, 32), ub.reshape(NP, 32),
                           da, db, batch_p, ones_p, zrows, zvec)

    pool = jnp.concatenate([pa, pb], axis=1)
    return pool + jnp.where(cnt > 0, 1.0, 0.0)[:, None] * b3[None, :]


# speedup vs baseline: 22.7337x; 2.3254x over previous
import functools

import jax
import jax.numpy as jnp
from jax import lax
from jax.experimental import pallas as pl
from jax.experimental.pallas import tpu as pltpu
from jax.experimental.pallas import tpu_sc as plsc

N = 50000
IN = 128
F = 64
G = 64
E = 800000
EREAL = E + N
NP = 50176
EP = 851968
NSUB = 16
NPS = NP // NSUB
EPS = EP // NSUB
EC = 128
NEC = EPS // EC
NCHK = 112
BN = 448
GRID = NP // BN

_f32 = jnp.float32
_i32 = jnp.int32


def _tc1_body(x_ref, w_ref, a_ref, ha_ref, hb_ref, sd_ref):
    h = jnp.dot(x_ref[...], w_ref[...], preferred_element_type=_f32)
    ha_ref[...] = h[:, :32]
    hb_ref[...] = h[:, 32:]
    sd_ref[...] = jnp.dot(h, a_ref[...], preferred_element_type=_f32)


_tc1 = pl.pallas_call(
    _tc1_body,
    grid=(GRID,),
    in_specs=[
        pl.BlockSpec((BN, IN), lambda i: (i, 0)),
        pl.BlockSpec((IN, F), lambda i: (0, 0)),
        pl.BlockSpec((F, 2), lambda i: (0, 0)),
    ],
    out_specs=[
        pl.BlockSpec((BN, 32), lambda i: (i, 0)),
        pl.BlockSpec((BN, 32), lambda i: (i, 0)),
        pl.BlockSpec((BN, 2), lambda i: (i, 0)),
    ],
    out_shape=[
        jax.ShapeDtypeStruct((NP, 32), _f32),
        jax.ShapeDtypeStruct((NP, 32), _f32),
        jax.ShapeDtypeStruct((NP, 2), _f32),
    ],
)


def _norm_body(ua_ref, ub_ref, dna_ref, dnb_ref, b_ref, w_ref, a_ref,
               ha_ref, hb_ref, sd_ref):
    u = jnp.concatenate([ua_ref[...], ub_ref[...]], axis=1)
    dn = jnp.maximum(dna_ref[...] + dnb_ref[...], 1e-30)
    hin = jnp.maximum(u / dn + b_ref[...], 0.0)
    h = jnp.dot(hin, w_ref[...], preferred_element_type=_f32)
    ha_ref[...] = h[:, :32]
    hb_ref[...] = h[:, 32:]
    sd_ref[...] = jnp.dot(h, a_ref[...], preferred_element_type=_f32)


_tc_norm = pl.pallas_call(
    _norm_body,
    grid=(GRID,),
    in_specs=[
        pl.BlockSpec((BN, 32), lambda i: (i, 0)),
        pl.BlockSpec((BN, 32), lambda i: (i, 0)),
        pl.BlockSpec((BN, 1), lambda i: (i, 0)),
        pl.BlockSpec((BN, 1), lambda i: (i, 0)),
        pl.BlockSpec((1, F), lambda i: (0, 0)),
        pl.BlockSpec((F, F), lambda i: (0, 0)),
        pl.BlockSpec((F, 2), lambda i: (0, 0)),
    ],
    out_specs=[
        pl.BlockSpec((BN, 32), lambda i: (i, 0)),
        pl.BlockSpec((BN, 32), lambda i: (i, 0)),
        pl.BlockSpec((BN, 2), lambda i: (i, 0)),
    ],
    out_shape=[
        jax.ShapeDtypeStruct((NP, 32), _f32),
        jax.ShapeDtypeStruct((NP, 32), _f32),
        jax.ShapeDtypeStruct((NP, 2), _f32),
    ],
)


_mesh = plsc.VectorSubcoreMesh(core_axis_name="c", subcore_axis_name="s")

EPW = EP // 32
NWC = EPW // EC


def _iota16():
    return lax.broadcasted_iota(_i32, (16,), 0)


def _weights_pass(wid, sd_v, srcp, dstp, src_v, dst_v, w_v, w_out, den_s):
    iota = _iota16()

    def chunk(k, carry):
        base = wid * EPW + k * EC
        pltpu.sync_copy(srcp.at[pl.ds(base, EC)], src_v)
        pltpu.sync_copy(dstp.at[pl.ds(base, EC)], dst_v)
        for j in range(EC // 16):
            s16 = src_v[pl.ds(j * 16, 16)]
            d16 = dst_v[pl.ds(j * 16, 16)]
            a_s = plsc.load_gather(sd_v, [s16 + s16])
            a_d = plsc.load_gather(sd_v, [d16 + d16 + 1])
            e = a_s + a_d
            e = jnp.where(e > 0.0, e, 0.2 * e)
            w = jnp.exp(e)
            gid = base + (iota + j * 16)
            w = jnp.where(gid < EREAL, w, 0.0)
            w_v[pl.ds(j * 16, 16)] = w
        pltpu.sync_copy(w_v, w_out.at[pl.ds(base, EC)])
        pltpu.sync_copy(w_v, den_s.at[dst_v], add=True)
        return carry

    lax.fori_loop(0, NWC, chunk, 0)


def _sc_w_body(sd, srcp, dstp, zvec, w_out, dna, dnb,
               sd_v, src_v, dst_v, w_v, dbuf, den_s):
    cid = lax.axis_index("c")
    sid = lax.axis_index("s")
    wid = cid * NSUB + sid

    pltpu.sync_copy(zvec, dbuf)

    def zchunk(q, carry):
        pltpu.sync_copy(dbuf, den_s.at[pl.ds(sid * NPS + q * NCHK, NCHK)])
        return carry

    lax.fori_loop(0, NPS // NCHK, zchunk, 0)
    pltpu.sync_copy(sd, sd_v)
    plsc.subcore_barrier()
    _weights_pass(wid, sd_v, srcp, dstp, src_v, dst_v, w_v, w_out, den_s)
    plsc.subcore_barrier()

    def ochunk(q, carry):
        nbase = sid * NPS + q * NCHK
        pltpu.sync_copy(den_s.at[pl.ds(nbase, NCHK)], dbuf)

        @pl.when(cid == 0)
        def _():
            pltpu.sync_copy(dbuf, dna.at[pl.ds(nbase, NCHK)])

        @pl.when(cid == 1)
        def _():
            pltpu.sync_copy(dbuf, dnb.at[pl.ds(nbase, NCHK)])
        return carry

    lax.fori_loop(0, NPS // NCHK, ochunk, 0)


_sc_w = functools.partial(
    pl.kernel,
    out_type=(
        jax.ShapeDtypeStruct((EP,), _f32),
        jax.ShapeDtypeStruct((NP,), _f32),
        jax.ShapeDtypeStruct((NP,), _f32),
    ),
    mesh=_mesh,
    scratch_types=[
        pltpu.VMEM((2 * NP,), _f32),
        pltpu.VMEM((EC,), _i32),
        pltpu.VMEM((EC,), _i32),
        pltpu.VMEM((EC,), _f32),
        pltpu.VMEM((NCHK,), _f32),
        pltpu.VMEM_SHARED((NP,), _f32),
    ],
    compiler_params=pltpu.CompilerParams(needs_layout_passes=False, use_tc_tiling_on_sc=False),
)(_sc_w_body)


def _stage(k, srcp, dstp, w_hbm, sv, dv, wv, ssem, sid):
    base = sid * EPS + k * EC
    pltpu.async_copy(srcp.at[pl.ds(base, EC)], sv, ssem)
    pltpu.async_copy(dstp.at[pl.ds(base, EC)], dv, ssem)
    pltpu.async_copy(w_hbm.at[pl.ds(base, EC)], wv, ssem)


def _wait_stage(k, srcp, dstp, w_hbm, sv, dv, wv, ssem, sid):
    base = sid * EPS + k * EC
    pltpu.make_async_copy(srcp.at[pl.ds(base, EC)], sv, ssem).wait()
    pltpu.make_async_copy(dstp.at[pl.ds(base, EC)], dv, ssem).wait()
    pltpu.make_async_copy(w_hbm.at[pl.ds(base, EC)], wv, ssem).wait()


def _rows_pass(sid, table_hbm, w_hbm, srcp, dstp, bufs, u_s, scsem):
    def sub(k, b):
        sv, dv, wv, rv, cv, ssem, gsem = bufs[b]
        nb = 1 - b
        svn, dvn, wvn, _, _, ssemn, _ = bufs[nb]
        _wait_stage(k, srcp, dstp, w_hbm, sv, dv, wv, ssem, sid)
        pltpu.async_copy(table_hbm.at[sv], rv, gsem)

        @pl.when(k >= 1)
        def _():
            pltpu.make_async_copy(cv, u_s.at[dv], scsem).wait()
        k2 = jnp.minimum(k + 1, NEC - 1)
        _stage(k2, srcp, dstp, w_hbm, svn, dvn, wvn, ssemn, sid)
        pltpu.make_async_copy(table_hbm.at[sv], rv, gsem).wait()
        for j in range(EC // 16):
            w16 = wv[pl.ds(j * 16, 16)]
            for t in range(16):
                m = j * 16 + t
                wspl = w16.at[jnp.full((16,), t, _i32)].get(
                    mode="promise_in_bounds")
                cv[m, pl.ds(0, 16)] = rv[m, pl.ds(0, 16)] * wspl
                cv[m, pl.ds(16, 16)] = rv[m, pl.ds(16, 16)] * wspl
        pltpu.async_copy(cv, u_s.at[dv], scsem, add=True)

    _stage(0, srcp, dstp, w_hbm, bufs[0][0], bufs[0][1], bufs[0][2],
           bufs[0][5], sid)

    def pair(i, carry):
        sub(2 * i, 0)
        sub(2 * i + 1, 1)
        return carry

    lax.fori_loop(0, NEC // 2, pair, 0)
    pltpu.make_async_copy(bufs[1][4], u_s.at[bufs[1][1]], scsem).wait()
    _wait_stage(NEC - 1, srcp, dstp, w_hbm, bufs[0][0], bufs[0][1],
                bufs[0][2], bufs[0][5], sid)


def _sc_u_half(sid, htab, w_hbm, srcp, dstp, u_out,
               bufs, obuf, obuf1, u_s, scsem):
    z16 = jnp.zeros((16,), _f32)
    for m in range(NCHK):
        obuf[m, pl.ds(0, 16)] = z16
        obuf[m, pl.ds(16, 16)] = z16

    def zchunk(q, carry):
        pltpu.sync_copy(obuf, u_s.at[pl.ds(sid * NPS + q * NCHK, NCHK), :])
        return carry

    lax.fori_loop(0, NPS // NCHK, zchunk, 0)
    plsc.subcore_barrier()
    _rows_pass(sid, htab, w_hbm, srcp, dstp, bufs, u_s, scsem)
    plsc.subcore_barrier()

    def ochunk(q, carry):
        nbase = sid * NPS + q * NCHK
        pltpu.sync_copy(u_s.at[pl.ds(nbase, NCHK), :], obuf)
        for m in range(NCHK):
            obuf1[pl.ds(m * 32, 16)] = obuf[m, pl.ds(0, 16)]
            obuf1[pl.ds(m * 32 + 16, 16)] = obuf[m, pl.ds(16, 16)]
        pltpu.sync_copy(obuf1, u_out.at[pl.ds(nbase * 32, NCHK * 32)])
        return carry

    lax.fori_loop(0, NPS // NCHK, ochunk, 0)


def _make_sc_u():
    def body(htab, w_hbm, srcp, dstp, u_out,
             sv0, dv0, wv0, rv0, cv0, ssem0, gsem0,
             sv1, dv1, wv1, rv1, cv1, ssem1, gsem1,
             obuf, obuf1, u_s, scsem):
        sid = lax.axis_index("s")
        bufs = ((sv0, dv0, wv0, rv0, cv0, ssem0, gsem0),
                (sv1, dv1, wv1, rv1, cv1, ssem1, gsem1))
        _sc_u_half(sid, htab, w_hbm, srcp, dstp, u_out,
                   bufs, obuf, obuf1, u_s, scsem)

    return functools.partial(
        pl.kernel,
        out_type=jax.ShapeDtypeStruct((NP * 32,), _f32),
        mesh=plsc.VectorSubcoreMesh(core_axis_name="c",
                                    subcore_axis_name="s", num_cores=1),
        scratch_types=(
            [pltpu.VMEM((EC,), _i32), pltpu.VMEM((EC,), _i32),
             pltpu.VMEM((EC,), _f32), pltpu.VMEM((EC, 32), _f32),
             pltpu.VMEM((EC, 32), _f32), pltpu.SemaphoreType.DMA,
             pltpu.SemaphoreType.DMA] * 2
            + [pltpu.VMEM((NCHK, 32), _f32),
               pltpu.VMEM((NCHK * 32,), _f32),
               pltpu.VMEM_SHARED((NP, 32), _f32),
               pltpu.SemaphoreType.DMA]
        ),
        compiler_params=pltpu.CompilerParams(needs_layout_passes=False, use_tc_tiling_on_sc=False),
        name="sc_u",
    )(body)


_sc_u = _make_sc_u()


def _pool_pass(sid, table_hbm, batch_p, ones_p, dna, dnb,
               b_v, o_v, dn_v, dn2_v, rows_v, cnt_v, rcnt_v,
               pool_s, cnt_s):
    def cnt_chunk(k, carry):
        base = sid * NPS + k * NCHK
        pltpu.sync_copy(batch_p.at[pl.ds(base, NCHK)], b_v)
        pltpu.sync_copy(ones_p.at[pl.ds(base, NCHK)], o_v)
        pltpu.sync_copy(o_v, cnt_s.at[b_v], add=True)
        return carry

    lax.fori_loop(0, NPS // NCHK, cnt_chunk, 0)
    plsc.subcore_barrier()

    pltpu.sync_copy(cnt_s, cnt_v)
    for k in range(G // 16):
        cv = cnt_v[pl.ds(k * 16, 16)]
        rcnt_v[pl.ds(k * 16, 16)] = 1.0 / jnp.maximum(cv, 1.0)

    def row_chunk(k, carry):
        base = sid * NPS + k * NCHK
        pltpu.sync_copy(batch_p.at[pl.ds(base, NCHK)], b_v)
        pltpu.sync_copy(ones_p.at[pl.ds(base, NCHK)], o_v)
        pltpu.sync_copy(dna.at[pl.ds(base, NCHK)], dn_v)
        pltpu.sync_copy(dnb.at[pl.ds(base, NCHK)], dn2_v)
        pltpu.sync_copy(table_hbm.at[pl.ds(base, NCHK), :], rows_v)
        for j in range(NCHK // 16):
            b16 = b_v[pl.ds(j * 16, 16)]
            o16 = o_v[pl.ds(j * 16, 16)]
            d16 = dn_v[pl.ds(j * 16, 16)] + dn2_v[pl.ds(j * 16, 16)]
            rc = plsc.load_gather(rcnt_v, [b16])
            w = o16 * rc / jnp.maximum(d16, 1e-30)
            for t in range(16):
                m = j * 16 + t
                wspl = w.at[jnp.full((16,), t, _i32)].get(
                    mode="promise_in_bounds")
                rows_v[m, pl.ds(0, 16)] = rows_v[m, pl.ds(0, 16)] * wspl
                rows_v[m, pl.ds(16, 16)] = rows_v[m, pl.ds(16, 16)] * wspl
        pltpu.sync_copy(rows_v, pool_s.at[b_v], add=True)
        return carry

    lax.fori_loop(0, NPS // NCHK, row_chunk, 0)
    plsc.subcore_barrier()


def _sc_pool_half(sid, table, batch_p, ones_p, dna, dnb, zrows, zvec, p_out,
                  cnt_out, write_cnt, b_v, o_v, dn_v, dn2_v, rows_v, cnt_v,
                  rcnt_v, pv, pool_s, cnt_s):
    @pl.when(sid == 0)
    def _():
        pltpu.sync_copy(zrows, rows_v)
        pltpu.sync_copy(zvec, dn_v)
        pltpu.sync_copy(rows_v.at[pl.ds(0, G), :], pool_s)
        pltpu.sync_copy(dn_v.at[pl.ds(0, G)], cnt_s)
    plsc.subcore_barrier()
    _pool_pass(sid, table, batch_p, ones_p, dna, dnb, b_v, o_v, dn_v, dn2_v,
               rows_v, cnt_v, rcnt_v, pool_s, cnt_s)

    @pl.when(sid == 0)
    def _():
        pltpu.sync_copy(pool_s, pv)
        pltpu.sync_copy(pv, p_out)
        if write_cnt:
            pltpu.sync_copy(cnt_s, cnt_v)
            pltpu.sync_copy(cnt_v, cnt_out)


def _sc_pool_body(u3a, u3b, dna, dnb, batch_p, ones_p, zrows, zvec,
                  pa, pb, cnt_out,
                  b_v, o_v, dn_v, dn2_v, rows_v, cnt_v, rcnt_v, pv,
                  pool_s, cnt_s):
    cid = lax.axis_index("c")
    sid = lax.axis_index("s")

    @pl.when(cid == 0)
    def _():
        _sc_pool_half(sid, u3a, batch_p, ones_p, dna, dnb, zrows, zvec, pa,
                      cnt_out, True, b_v, o_v, dn_v, dn2_v, rows_v, cnt_v,
                      rcnt_v, pv, pool_s, cnt_s)

    @pl.when(cid == 1)
    def _():
        _sc_pool_half(sid, u3b, batch_p, ones_p, dna, dnb, zrows, zvec, pb,
                      cnt_out, False, b_v, o_v, dn_v, dn2_v, rows_v, cnt_v,
                      rcnt_v, pv, pool_s, cnt_s)


_sc_pool = functools.partial(
    pl.kernel,
    out_type=(
        jax.ShapeDtypeStruct((G, 32), _f32),
        jax.ShapeDtypeStruct((G, 32), _f32),
        jax.ShapeDtypeStruct((G,), _f32),
    ),
    mesh=_mesh,
    scratch_types=[
        pltpu.VMEM((NCHK,), _i32),
        pltpu.VMEM((NCHK,), _f32),
        pltpu.VMEM((NCHK,), _f32),
        pltpu.VMEM((NCHK,), _f32),
        pltpu.VMEM((NCHK, 32), _f32),
        pltpu.VMEM((G,), _f32),
        pltpu.VMEM((G,), _f32),
        pltpu.VMEM((G, 32), _f32),
        pltpu.VMEM_SHARED((G, 32), _f32),
        pltpu.VMEM_SHARED((G,), _f32),
    ],
    compiler_params=pltpu.CompilerParams(needs_layout_passes=False, use_tc_tiling_on_sc=False),
)(_sc_pool_body)


def kernel(x, edge_index, batch, W1, a_src1, a_dst1, b1,
           W2, a_src2, a_dst2, b2, W3, a_src3, a_dst3, b3):
    loop = jnp.arange(N, dtype=edge_index.dtype)
    srcp = jnp.zeros((EP,), _i32).at[:EREAL].set(
        jnp.concatenate([edge_index[0], loop]).astype(_i32))
    dstp = jnp.zeros((EP,), _i32).at[:EREAL].set(
        jnp.concatenate([edge_index[1], loop]).astype(_i32))
    x_p = jnp.zeros((NP, IN), _f32).at[:N].set(x)
    batch_p = jnp.zeros((NP,), _i32).at[:N].set(batch.astype(_i32))
    ones_p = (jnp.arange(NP) < N).astype(_f32)
    zrows = jnp.zeros((NCHK, 32), _f32)
    zvec = jnp.zeros((NCHK,), _f32)

    a1 = jnp.stack([a_src1, a_dst1], axis=1)
    a2 = jnp.stack([a_src2, a_dst2], axis=1)
    a3 = jnp.stack([a_src3, a_dst3], axis=1)

    ha, hb, sd = _tc1(x_p, W1, a1)
    wv, da, db = _sc_w(sd.reshape(2 * NP), srcp, dstp, zvec)
    ua = _sc_u(ha, wv, srcp, dstp)
    ub = _sc_u(hb, wv, srcp, dstp)
    ha, hb, sd = _tc_norm(ua.reshape(NP, 32), ub.reshape(NP, 32),
                      da.reshape(NP, 1), db.reshape(NP, 1),
                      b1.reshape(1, F), W2, a2)
    wv, da, db = _sc_w(sd.reshape(2 * NP), srcp, dstp, zvec)
    ua = _sc_u(ha, wv, srcp, dstp)
    ub = _sc_u(hb, wv, srcp, dstp)
    ha, hb, sd = _tc_norm(ua.reshape(NP, 32), ub.reshape(NP, 32),
                      da.reshape(NP, 1), db.reshape(NP, 1),
                      b2.reshape(1, F), W3, a3)
    wv, da, db = _sc_w(sd.reshape(2 * NP), srcp, dstp, zvec)
    ua = _sc_u(ha, wv, srcp, dstp)
    ub = _sc_u(hb, wv, srcp, dstp)
    pa, pb, cnt = _sc_pool(ua.reshape(NP, 32), ub.reshape(NP, 32),
                           da, db, batch_p, ones_p, zrows, zvec)

    pool = jnp.concatenate([pa, pb], axis=1)
    return pool + jnp.where(cnt > 0, 1.0, 0.0)[:, None] * b3[None, :]
